# baseline (speedup 1.0000x reference)
import jax
import jax.numpy as jnp
from jax import lax
from jax.experimental import pallas as pl
from jax.experimental.pallas import tpu as pltpu
from jax.experimental.pallas import tpu_sc as plsc

BATCH = 16384
NROWS = 1000000
D0, D1, D2 = 16, 32, 64
_SEG = ((D0, 0), (D1, D0), (D2, D0 + D1))

_NC = 2
_NS = 16
_NW = _NC * _NS
_BPW = BATCH // _NW
_CHUNK = 128
_NCHUNK = _BPW // _CHUNK
_OSTRIDE = D0 + D1 + D2


def _sc_body(idx_hbm, w0, w1, w2, o0, o1, o2,
             idx_v, gbuf, obuf, sa, sb, soa, sob):
    wid = lax.axis_index("s") * _NC + lax.axis_index("c")
    base = wid * _BPW

    pltpu.sync_copy(idx_hbm.at[pl.ds(base, _BPW)], idx_v)

    gsems = (sa, sb)
    osems = (soa, sob)

    work = []
    for w, o, (d, _) in zip((w0, w1, w2), (o0, o1, o2), _SEG):
        for j in range(_NCHUNK):
            work.append((w, o, d, j))

    def fire(item, slot):
        w, _, _, j = item
        return pltpu.async_copy(
            w.at[idx_v.at[pl.ds(j * _CHUNK, _CHUNK)]],
            gbuf.at[slot], gsems[slot])

    def extract(item, slot):
        _, o, d, j = item

        def body(k, carry):
            for c in range(d // 16):
                v = gbuf[slot, k, pl.ds(c * 16, 16)]
                obuf[slot, pl.ds(k * d + c * 16, 16)] = jnp.maximum(v, 0.0)
            return carry

        lax.fori_loop(0, _CHUNK, body, 0)
        return pltpu.async_copy(
            obuf.at[slot, pl.ds(0, _CHUNK * d)],
            o.at[pl.ds((base + j * _CHUNK) * d, _CHUNK * d)], osems[slot])

    copies = [fire(work[0], 0), fire(work[1], 1)]
    outs = [None, None]
    for n, item in enumerate(work):
        slot = n % 2
        copies[n].wait()
        if outs[slot] is not None:
            outs[slot].wait()
        outs[slot] = extract(item, slot)
        if n + 2 < len(work):
            copies.append(fire(work[n + 2], slot))
    outs[0].wait()
    outs[1].wait()


def _obuf_layout_note():
    pass


_gather_relu = pl.kernel(
    _sc_body,
    out_type=(
        jax.ShapeDtypeStruct((BATCH * D0,), jnp.float32),
        jax.ShapeDtypeStruct((BATCH * D1,), jnp.float32),
        jax.ShapeDtypeStruct((BATCH * D2,), jnp.float32),
    ),
    mesh=plsc.VectorSubcoreMesh(core_axis_name="c", subcore_axis_name="s"),
    compiler_params=pltpu.CompilerParams(
        use_tc_tiling_on_sc=True, needs_layout_passes=False),
    scratch_types=[
        pltpu.VMEM((_BPW,), jnp.int32),
        pltpu.VMEM((2, _CHUNK, 128), jnp.float32),
        pltpu.VMEM((2, _CHUNK * D2), jnp.float32),
        pltpu.SemaphoreType.DMA,
        pltpu.SemaphoreType.DMA,
        pltpu.SemaphoreType.DMA,
        pltpu.SemaphoreType.DMA,
    ],
)


def kernel(indices, W0, W1, W2):
    idx = indices.astype(jnp.int32)
    p0 = jnp.pad(W0, ((0, 0), (0, 128 - D0)))
    p1 = jnp.pad(W1, ((0, 0), (0, 128 - D1)))
    p2 = jnp.pad(W2, ((0, 0), (0, 128 - D2)))
    f0, f1, f2 = _gather_relu(idx, p0, p1, p2)
    return (f0.reshape(BATCH, D0), f1.reshape(BATCH, D1),
            f2.reshape(BATCH, D2))


# speedup vs baseline: 1.1934x; 1.1934x over previous
import jax
import jax.numpy as jnp
from jax import lax
from jax.experimental import pallas as pl
from jax.experimental.pallas import tpu as pltpu
from jax.experimental.pallas import tpu_sc as plsc

BATCH = 16384
NROWS = 1000000
D0, D1, D2 = 16, 32, 64
_SEG = ((D0, 0), (D1, D0), (D2, D0 + D1))

_NC = 2
_NS = 16
_NW = _NC * _NS
_BPW = BATCH // _NW
_CHUNK = 128
_NCHUNK = _BPW // _CHUNK
_OSTRIDE = D0 + D1 + D2


def _sc_body(idx_hbm, wcat, o0, o1, o2, idx_v, gbuf, obuf, sa, sb, soa, sob):
    wid = lax.axis_index("s") * _NC + lax.axis_index("c")
    base = wid * _BPW

    pltpu.sync_copy(idx_hbm.at[pl.ds(base, _BPW)], idx_v)

    gsems = (sa, sb)
    osems = (soa, sob)

    def fire(j, slot):
        return pltpu.async_copy(
            wcat.at[idx_v.at[pl.ds(j * _CHUNK, _CHUNK)]],
            gbuf.at[slot], gsems[slot])

    def extract(j, slot):
        def body(k, carry):
            pos = 0
            for (d, col) in _SEG:
                for c in range(d // 16):
                    v = gbuf[slot, k, pl.ds(col + c * 16, 16)]
                    obuf[slot, pl.ds(pos * _CHUNK + k * d + c * 16, 16)] = (
                        jnp.maximum(v, 0.0))
                pos += d
            return carry

        lax.fori_loop(0, _CHUNK, body, 0)
        outs = []
        pos = 0
        for (d, _), o in zip(_SEG, (o0, o1, o2)):
            outs.append(pltpu.async_copy(
                obuf.at[slot, pl.ds(pos * _CHUNK, _CHUNK * d)],
                o.at[pl.ds((base + j * _CHUNK) * d, _CHUNK * d)],
                osems[slot]))
            pos += d
        return outs

    copies = [fire(0, 0), fire(1, 1)]
    outs = [None, None]
    for j in range(_NCHUNK):
        slot = j % 2
        copies[j].wait()
        if outs[slot] is not None:
            for c in outs[slot]:
                c.wait()
        outs[slot] = extract(j, slot)
        if j + 2 < _NCHUNK:
            copies.append(fire(j + 2, slot))
    for group in outs:
        for c in group:
            c.wait()


def _obuf_layout_note():
    pass


_gather_relu = pl.kernel(
    _sc_body,
    out_type=(
        jax.ShapeDtypeStruct((BATCH * D0,), jnp.float32),
        jax.ShapeDtypeStruct((BATCH * D1,), jnp.float32),
        jax.ShapeDtypeStruct((BATCH * D2,), jnp.float32),
    ),
    mesh=plsc.VectorSubcoreMesh(core_axis_name="c", subcore_axis_name="s"),
    compiler_params=pltpu.CompilerParams(
        use_tc_tiling_on_sc=True, needs_layout_passes=False),
    scratch_types=[
        pltpu.VMEM((_BPW,), jnp.int32),
        pltpu.VMEM((2, _CHUNK, 128), jnp.float32),
        pltpu.VMEM((2, _CHUNK * _OSTRIDE), jnp.float32),
        pltpu.SemaphoreType.DMA,
        pltpu.SemaphoreType.DMA,
        pltpu.SemaphoreType.DMA,
        pltpu.SemaphoreType.DMA,
    ],
)


def kernel(indices, W0, W1, W2):
    idx = indices.astype(jnp.int32)
    wcat = jnp.pad(jnp.concatenate((W0, W1, W2), axis=1), ((0, 0), (0, 16)))
    f0, f1, f2 = _gather_relu(idx, wcat)
    return (f0.reshape(BATCH, D0), f1.reshape(BATCH, D1),
            f2.reshape(BATCH, D2))
